# baseline (speedup 1.0000x reference)
import jax
import jax.numpy as jnp
from jax.experimental import pallas as pl
from jax.experimental.pallas import tpu as pltpu

NUM_HEADS = 12
SEQ = 256
HEAD_DIM = 64
LANES = 1024
SUB = 16


def _copy_kernel(rev_ref, out_ref):
    m = pl.program_id(0)
    y = pltpu.roll(rev_ref[...], (m + 17) % 32, axis=1)

    out_ref[:, 0, 0] = y[:, 1 : SUB + 1, :]
    for ib in range(1, 16):
        r = HEAD_DIM * (16 - ib)
        rr = pltpu.roll(y[:, 0 : SUB + 1, :], LANES - r, axis=2)
        mask = (
            jax.lax.broadcasted_iota(
                jnp.int32, (NUM_HEADS, SUB, LANES), 2
            )
            < LANES - r
        )
        out_ref[:, 0, ib] = jnp.where(
            mask, rr[:, 0:SUB, :], rr[:, 1 : SUB + 1, :]
        )


def kernel(seq_length, relative_positional_encoding):
    start = seq_length - SEQ
    sl = jax.lax.dynamic_slice(
        relative_positional_encoding,
        (0, start, 0),
        (NUM_HEADS, 2 * SEQ, HEAD_DIM),
    )
    rev = sl[:, ::-1, :].reshape(NUM_HEADS, 2 * SEQ * HEAD_DIM // LANES, LANES)

    out = pl.pallas_call(
        _copy_kernel,
        grid=(16,),
        in_specs=[
            pl.BlockSpec(rev.shape, lambda m: (0, 0, 0)),
        ],
        out_specs=pl.BlockSpec(
            (NUM_HEADS, 1, 16, SUB, LANES), lambda m: (0, m, 0, 0, 0)
        ),
        out_shape=jax.ShapeDtypeStruct(
            (NUM_HEADS, 16, 16, SUB, LANES), jnp.float32
        ),
    )(rev)
    return out.reshape(NUM_HEADS, SEQ, SEQ, HEAD_DIM)


# speedup vs baseline: 6.5130x; 6.5130x over previous
import jax
import jax.numpy as jnp
from jax.experimental import pallas as pl
from jax.experimental.pallas import tpu as pltpu

NUM_HEADS = 12
SEQ = 256
HEAD_DIM = 64
BPS = 8


def _copy_kernel(revt_ref, out_ref):
    g = pl.program_id(0)
    revt = revt_ref[...]
    for db in range(BPS):
        b = BPS * g + db + 1
        rolled = pltpu.roll(revt, (2 * SEQ - b) % (2 * SEQ), axis=2)
        out_ref[:, 1, BPS - 1 - db] = rolled[:, :, 0:SEQ]
        out_ref[:, 0, BPS - 1 - db] = rolled[:, :, SEQ // 2 : 3 * SEQ // 2]


def kernel(seq_length, relative_positional_encoding):
    start = seq_length - SEQ
    sl = jax.lax.dynamic_slice(
        relative_positional_encoding,
        (0, start, 0),
        (NUM_HEADS, 2 * SEQ, HEAD_DIM),
    )
    revt = sl[:, ::-1, :].transpose(0, 2, 1)

    out = pl.pallas_call(
        _copy_kernel,
        grid=(SEQ // 2 // BPS,),
        in_specs=[
            pl.BlockSpec(
                (NUM_HEADS, HEAD_DIM, 2 * SEQ), lambda g: (0, 0, 0)
            ),
        ],
        out_specs=pl.BlockSpec(
            (NUM_HEADS, 2, BPS, HEAD_DIM, SEQ),
            lambda g: (0, 0, (SEQ // 2 // BPS) - 1 - g, 0, 0),
        ),
        out_shape=jax.ShapeDtypeStruct(
            (NUM_HEADS, 2, SEQ // 2, HEAD_DIM, SEQ), jnp.float32
        ),
    )(revt)
    return out.reshape(NUM_HEADS, SEQ, HEAD_DIM, SEQ).transpose(0, 1, 3, 2)


# speedup vs baseline: 6.9826x; 1.0721x over previous
import jax
import jax.numpy as jnp
from jax.experimental import pallas as pl
from jax.experimental.pallas import tpu as pltpu

NUM_HEADS = 12
SEQ = 256
HEAD_DIM = 64
BPS = 8


def _copy_kernel(revt_ref, out_ref):
    g = pl.program_id(0)
    revt = revt_ref[...]
    base = pltpu.roll(revt, (2 * SEQ - (BPS * g + 1)) % (2 * SEQ), axis=2)
    for db in range(BPS):
        rolled = pltpu.roll(base, 2 * SEQ - db, axis=2) if db else base
        out_ref[:, 1, BPS - 1 - db] = rolled[:, :, 0:SEQ]
        out_ref[:, 0, BPS - 1 - db] = rolled[:, :, SEQ // 2 : 3 * SEQ // 2]


def kernel(seq_length, relative_positional_encoding):
    start = seq_length - SEQ
    sl = jax.lax.dynamic_slice(
        relative_positional_encoding,
        (0, start, 0),
        (NUM_HEADS, 2 * SEQ, HEAD_DIM),
    )
    revt = sl[:, ::-1, :].transpose(0, 2, 1)

    out = pl.pallas_call(
        _copy_kernel,
        grid=(SEQ // 2 // BPS,),
        in_specs=[
            pl.BlockSpec(
                (NUM_HEADS, HEAD_DIM, 2 * SEQ), lambda g: (0, 0, 0)
            ),
        ],
        out_specs=pl.BlockSpec(
            (NUM_HEADS, 2, BPS, HEAD_DIM, SEQ),
            lambda g: (0, 0, (SEQ // 2 // BPS) - 1 - g, 0, 0),
        ),
        out_shape=jax.ShapeDtypeStruct(
            (NUM_HEADS, 2, SEQ // 2, HEAD_DIM, SEQ), jnp.float32
        ),
    )(revt)
    return out.reshape(NUM_HEADS, SEQ, HEAD_DIM, SEQ).transpose(0, 1, 3, 2)
